# baseline (speedup 1.0000x reference)
import jax
import jax.numpy as jnp
from jax import lax
from jax.experimental import pallas as pl
from jax.experimental.pallas import tpu as pltpu
from jax.experimental.pallas import tpu_sc as plsc

B = 16
S = 2048
F = 32
H = 64
SE = S - 1
NT = SE - H + 1
NC = 16
NQ = NT // NC
DEC_F = 8
DEC_W = H * DEC_F


def _fork_body(x_ref, m_ref, xr_ref, enc_ref, dec_ref, his_ref):
    enc_ref[0] = x_ref[0, :SE, :24]
    his_ref[0] = m_ref[0, :SE, :]

    A = xr_ref[0]
    for r in range(NC):
        s = 8 * r + 8
        pieces = []
        for k in range(4):
            if s == 128:
                pieces.append(A[k + 1:k + 1 + NQ])
            else:
                a1 = A[k:k + NQ]
                a2 = A[k + 1:k + 1 + NQ]
                pieces.append(
                    jnp.concatenate([a1[:, s:], a2[:, :s]], axis=1))
        cls = jnp.concatenate(pieces, axis=1)
        for q in range(NQ):
            t = NC * q + r
            dec_ref[0, t:t + 1, :] = cls[q:q + 1]

NU = 32
HALF = NT // 2
FCH = 32
_SC_MESH = plsc.VectorSubcoreMesh(core_axis_name="c", subcore_axis_name="s")


def _fut_sc(mf):
    @pl.kernel(
        mesh=_SC_MESH,
        out_type=jax.ShapeDtypeStruct((B, 2, HALF, H), jnp.float32),
        scratch_types=[
            pltpu.VMEM((S,), jnp.float32),
            pltpu.VMEM((FCH, H), jnp.float32),
            pltpu.SemaphoreType.DMA,
        ],
    )
    def k(mf_hbm, fut_hbm, m_v, o_v, sem):
        wid = lax.axis_index("c") * 16 + lax.axis_index("s")
        b = wid // 2
        half = wid % 2
        t_base = half * HALF
        pltpu.async_copy(mf_hbm.at[b], m_v, sem).wait()

        @pl.loop(0, HALF // FCH)
        def _(ch):
            @pl.loop(0, FCH)
            def _(i):
                st = t_base + ch * FCH + i + 1
                for j in range(H // 16):
                    o_v[i, pl.ds(16 * j, 16)] = m_v[pl.ds(st + 16 * j, 16)]
            pltpu.async_copy(
                o_v, fut_hbm.at[b, half, pl.ds(ch * FCH, FCH), :], sem
            ).wait()

    return k(mf)


def kernel(tensor, masking):
    xr = tensor[:, :, 24:32].reshape(B, 128, 128)
    enc, dec4, his = pl.pallas_call(
        _fork_body,
        grid=(B,),
        in_specs=[
            pl.BlockSpec((1, S, F), lambda b: (b, 0, 0)),
            pl.BlockSpec((1, S, 1), lambda b: (b, 0, 0)),
            pl.BlockSpec((1, 128, 128), lambda b: (b, 0, 0)),
        ],
        out_specs=[
            pl.BlockSpec((1, SE, 24), lambda b: (b, 0, 0)),
            pl.BlockSpec((1, NT, DEC_W), lambda b: (b, 0, 0)),
            pl.BlockSpec((1, SE, 1), lambda b: (b, 0, 0)),
        ],
        out_shape=[
            jax.ShapeDtypeStruct((B, SE, 24), jnp.float32),
            jax.ShapeDtypeStruct((B, NT, DEC_W), jnp.float32),
            jax.ShapeDtypeStruct((B, SE, 1), jnp.float32),
        ],
        compiler_params=pltpu.CompilerParams(
            dimension_semantics=("parallel",),
        ),
    )(tensor, masking, xr)
    fut4 = _fut_sc(masking.reshape(B, S))
    dec = dec4.reshape(B, NT, H, DEC_F)
    fut = fut4.reshape(B, NT, H, 1)
    return (enc, dec, his, fut)


# speedup vs baseline: 1.0312x; 1.0312x over previous
import jax
import jax.numpy as jnp
from jax import lax
from jax.experimental import pallas as pl
from jax.experimental.pallas import tpu as pltpu
from jax.experimental.pallas import tpu_sc as plsc

B = 16
S = 2048
F = 32
H = 64
SE = S - 1
NT = SE - H + 1
NC = 16
NQ = NT // NC
DEC_F = 8
DEC_W = H * DEC_F


def _fork_body(x_ref, m_ref, xr_ref, enc_ref, dec_ref, his_ref):
    enc_ref[0] = x_ref[0, :SE, :24]
    his_ref[0] = m_ref[0, :SE, :]

    A = xr_ref[0]
    for r in range(NC):
        s = 8 * r + 8
        pieces = []
        for k in range(4):
            if s == 128:
                pieces.append(A[k + 1:k + 1 + NQ])
            else:
                a1 = A[k:k + NQ]
                a2 = A[k + 1:k + 1 + NQ]
                pieces.append(
                    jnp.concatenate([a1[:, s:], a2[:, :s]], axis=1))
        cls = jnp.concatenate(pieces, axis=1)
        for q in range(NQ):
            t = NC * q + r
            dec_ref[0, t:t + 1, :] = cls[q:q + 1]

HALF = NT // 2
FCH = 16
_SC_MESH = plsc.VectorSubcoreMesh(core_axis_name="c", subcore_axis_name="s")


def _fut_sc(mf):
    @pl.kernel(
        mesh=_SC_MESH,
        out_type=jax.ShapeDtypeStruct((B, 2, HALF, H), jnp.float32),
        scratch_types=[
            pltpu.VMEM((S,), jnp.float32),
            pltpu.VMEM((FCH, H), jnp.float32),
            pltpu.VMEM((FCH, H), jnp.float32),
            pltpu.SemaphoreType.DMA,
            pltpu.SemaphoreType.DMA,
        ],
    )
    def k(mf_hbm, fut_hbm, m_v, o_v0, o_v1, sem0, sem1):
        wid = lax.axis_index("c") * 16 + lax.axis_index("s")
        b = wid // 2
        half = wid % 2
        t_base = half * HALF
        pltpu.async_copy(mf_hbm.at[b], m_v, sem0).wait()

        def fill(o_v, ch):
            @pl.loop(0, FCH)
            def _(i):
                st = t_base + ch * FCH + i + 1
                for j in range(H // 16):
                    o_v[i, pl.ds(16 * j, 16)] = m_v[pl.ds(st + 16 * j, 16)]

        def dst(ch):
            return fut_hbm.at[b, half, pl.ds(ch * FCH, FCH), :]

        NCH = HALF // FCH

        @pl.loop(0, NCH, step=2)
        def _(ch):
            @pl.when(ch > 0)
            def _():
                pltpu.make_async_copy(o_v0, dst(ch - 2), sem0).wait()
            fill(o_v0, ch)
            pltpu.async_copy(o_v0, dst(ch), sem0)

            @pl.when(ch > 0)
            def _():
                pltpu.make_async_copy(o_v1, dst(ch - 1), sem1).wait()
            fill(o_v1, ch + 1)
            pltpu.async_copy(o_v1, dst(ch + 1), sem1)

        pltpu.make_async_copy(o_v0, dst(NCH - 2), sem0).wait()
        pltpu.make_async_copy(o_v1, dst(NCH - 1), sem1).wait()

    return k(mf)


def kernel(tensor, masking):
    xr = tensor[:, :, 24:32].reshape(B, 128, 128)
    enc, dec4, his = pl.pallas_call(
        _fork_body,
        grid=(B,),
        in_specs=[
            pl.BlockSpec((1, S, F), lambda b: (b, 0, 0)),
            pl.BlockSpec((1, S, 1), lambda b: (b, 0, 0)),
            pl.BlockSpec((1, 128, 128), lambda b: (b, 0, 0)),
        ],
        out_specs=[
            pl.BlockSpec((1, SE, 24), lambda b: (b, 0, 0)),
            pl.BlockSpec((1, NT, DEC_W), lambda b: (b, 0, 0)),
            pl.BlockSpec((1, SE, 1), lambda b: (b, 0, 0)),
        ],
        out_shape=[
            jax.ShapeDtypeStruct((B, SE, 24), jnp.float32),
            jax.ShapeDtypeStruct((B, NT, DEC_W), jnp.float32),
            jax.ShapeDtypeStruct((B, SE, 1), jnp.float32),
        ],
        compiler_params=pltpu.CompilerParams(
            dimension_semantics=("parallel",),
        ),
    )(tensor, masking, xr)
    fut4 = _fut_sc(masking.reshape(B, S))
    dec = dec4.reshape(B, NT, H, DEC_F)
    fut = fut4.reshape(B, NT, H, 1)
    return (enc, dec, his, fut)
